# baseline (speedup 1.0000x reference)
import math

import jax
import jax.numpy as jnp
from jax.experimental import pallas as pl
from jax.experimental.pallas import tpu as pltpu

_SIZE = 32000
_N_TOK = 4096
_SMOOTHING = 0.1
_CONF = 1.0 - _SMOOTHING
_EPS = _SMOOTHING / (_SIZE - 2)
_C_ROW = (_SIZE - 2) * _EPS * math.log(_EPS) + _CONF * math.log(_CONF)

_BR = 256
_BC = 6400


def _loss_body(x_ref, t_ref, wf_ref, wc_ref, mf_ref, out_ref):
    i = pl.program_id(0)
    j = pl.program_id(1)

    @pl.when((i == 0) & (j == 0))
    def _init():
        out_ref[0, 0] = 0.0

    x = x_ref[...]
    t = t_ref[...]
    col = jax.lax.broadcasted_iota(jnp.int32, (_BR, _BC), 1) + j * _BC
    w = jnp.where(col == t, wc_ref[...], wf_ref[...])
    out_ref[0, 0] += jnp.sum(x * w)

    @pl.when(j == 0)
    def _col0():
        out_ref[0, 0] += jnp.sum(mf_ref[...] * (_C_ROW + _EPS * x[:, 0:1]))


def kernel(x, target):
    t2 = target.reshape(_N_TOK, 1)
    maskf = (t2 != 0).astype(jnp.float32)
    wf = -_EPS * maskf
    wc = -_CONF * maskf
    out = pl.pallas_call(
        _loss_body,
        grid=(_N_TOK // _BR, _SIZE // _BC),
        in_specs=[
            pl.BlockSpec((_BR, _BC), lambda i, j: (i, j)),
            pl.BlockSpec((_BR, 1), lambda i, j: (i, 0)),
            pl.BlockSpec((_BR, 1), lambda i, j: (i, 0)),
            pl.BlockSpec((_BR, 1), lambda i, j: (i, 0)),
            pl.BlockSpec((_BR, 1), lambda i, j: (i, 0)),
        ],
        out_specs=pl.BlockSpec((1, 1), lambda i, j: (0, 0),
                               memory_space=pltpu.SMEM),
        out_shape=jax.ShapeDtypeStruct((1, 1), jnp.float32),
    )(x, t2, wf, wc, maskf)
    return out[0, 0]


# speedup vs baseline: 1.1527x; 1.1527x over previous
import math

import jax
import jax.numpy as jnp
from jax.experimental import pallas as pl
from jax.experimental.pallas import tpu as pltpu

_SIZE = 32000
_N_TOK = 4096
_SMOOTHING = 0.1
_CONF = 1.0 - _SMOOTHING
_EPS = _SMOOTHING / (_SIZE - 2)
_C_ROW = (_SIZE - 2) * _EPS * math.log(_EPS) + _CONF * math.log(_CONF)

_BR = 256
_BC = 16000


def _loss_body(x_ref, t_ref, wf_ref, wc_ref, mf_ref, out_ref):
    i = pl.program_id(0)
    j = pl.program_id(1)

    @pl.when((i == 0) & (j == 0))
    def _init():
        out_ref[0, 0] = 0.0

    x = x_ref[...]
    t = t_ref[...]
    col = jax.lax.broadcasted_iota(jnp.int32, (_BR, _BC), 1) + j * _BC
    w = jnp.where(col == t, wc_ref[...], wf_ref[...])
    out_ref[0, 0] += jnp.sum(x * w)

    @pl.when(j == 0)
    def _col0():
        out_ref[0, 0] += jnp.sum(mf_ref[...] * (_C_ROW + _EPS * x[:, 0:1]))


def kernel(x, target):
    t2 = target.reshape(_N_TOK, 1)
    maskf = (t2 != 0).astype(jnp.float32)
    wf = -_EPS * maskf
    wc = -_CONF * maskf
    out = pl.pallas_call(
        _loss_body,
        grid=(_N_TOK // _BR, _SIZE // _BC),
        in_specs=[
            pl.BlockSpec((_BR, _BC), lambda i, j: (i, j)),
            pl.BlockSpec((_BR, 1), lambda i, j: (i, 0)),
            pl.BlockSpec((_BR, 1), lambda i, j: (i, 0)),
            pl.BlockSpec((_BR, 1), lambda i, j: (i, 0)),
            pl.BlockSpec((_BR, 1), lambda i, j: (i, 0)),
        ],
        out_specs=pl.BlockSpec((1, 1), lambda i, j: (0, 0),
                               memory_space=pltpu.SMEM),
        out_shape=jax.ShapeDtypeStruct((1, 1), jnp.float32),
    )(x, t2, wf, wc, maskf)
    return out[0, 0]


# speedup vs baseline: 1.1880x; 1.0306x over previous
import math

import jax
import jax.numpy as jnp
from jax.experimental import pallas as pl
from jax.experimental.pallas import tpu as pltpu

_SIZE = 32000
_N_TOK = 4096
_SMOOTHING = 0.1
_CONF = 1.0 - _SMOOTHING
_EPS = _SMOOTHING / (_SIZE - 2)
_C_ROW = (_SIZE - 2) * _EPS * math.log(_EPS) + _CONF * math.log(_CONF)

_BR = 128
_BC = 32000


def _loss_body(x_ref, t_ref, wf_ref, wc_ref, mf_ref, out_ref):
    i = pl.program_id(0)
    j = pl.program_id(1)

    @pl.when((i == 0) & (j == 0))
    def _init():
        out_ref[0, 0] = 0.0

    x = x_ref[...]
    t = t_ref[...]
    col = jax.lax.broadcasted_iota(jnp.int32, (_BR, _BC), 1) + j * _BC
    w = jnp.where(col == t, wc_ref[...], wf_ref[...])
    out_ref[0, 0] += jnp.sum(x * w)

    @pl.when(j == 0)
    def _col0():
        out_ref[0, 0] += jnp.sum(mf_ref[...] * (_C_ROW + _EPS * x[:, 0:1]))


def kernel(x, target):
    t2 = target.reshape(_N_TOK, 1)
    maskf = (t2 != 0).astype(jnp.float32)
    wf = -_EPS * maskf
    wc = -_CONF * maskf
    out = pl.pallas_call(
        _loss_body,
        grid=(_N_TOK // _BR, _SIZE // _BC),
        in_specs=[
            pl.BlockSpec((_BR, _BC), lambda i, j: (i, j)),
            pl.BlockSpec((_BR, 1), lambda i, j: (i, 0)),
            pl.BlockSpec((_BR, 1), lambda i, j: (i, 0)),
            pl.BlockSpec((_BR, 1), lambda i, j: (i, 0)),
            pl.BlockSpec((_BR, 1), lambda i, j: (i, 0)),
        ],
        out_specs=pl.BlockSpec((1, 1), lambda i, j: (0, 0),
                               memory_space=pltpu.SMEM),
        out_shape=jax.ShapeDtypeStruct((1, 1), jnp.float32),
    )(x, t2, wf, wc, maskf)
    return out[0, 0]
